# baseline (speedup 1.0000x reference)
import jax
import jax.numpy as jnp
from jax import lax
from jax.experimental import pallas as pl
from jax.experimental.pallas import tpu as pltpu
from jax.experimental.pallas import tpu_sc as plsc

NUM_TOKENS = 1000000
DIM = 32
BATCH = 16384
HIST = 50

_N = BATCH * HIST
_NW = 32
_PER_W = _N // _NW
_C = 640
_NBUF = 4
_NSTEPS = _PER_W // _C


def _emb_body(table, xflat, out, idx_v, *bufs):
    rows = bufs[:_NBUF]
    gs = bufs[_NBUF:2 * _NBUF]
    os_ = bufs[2 * _NBUF:3 * _NBUF]
    c = lax.axis_index("c")
    s = lax.axis_index("s")
    wid = s * 2 + c
    base = wid * _PER_W
    pltpu.sync_copy(xflat.at[pl.ds(base, _PER_W)], idx_v)

    gh = [None] * _NBUF
    oh = [None] * _NBUF

    def start(step, b):
        off = step * _C
        gh[b] = pltpu.async_copy(table.at[idx_v.at[pl.ds(off, _C)]],
                                 rows[b], gs[b])

    for j in range(_NBUF):
        start(j, j)
    for i in range(_NSTEPS):
        b = i % _NBUF
        gh[b].wait()
        oh[b] = pltpu.async_copy(rows[b], out.at[pl.ds(base + i * _C, _C)],
                                 os_[b])
        nxt = i + _NBUF
        if nxt < _NSTEPS:
            oh[b].wait()
            start(nxt, b)
    for j in range(max(0, _NSTEPS - _NBUF), _NSTEPS):
        oh[j % _NBUF].wait()


@jax.jit
def _emb(xflat, table):
    mesh = plsc.VectorSubcoreMesh(core_axis_name="c", subcore_axis_name="s")
    f = pl.kernel(
        _emb_body,
        mesh=mesh,
        compiler_params=pltpu.CompilerParams(use_tc_tiling_on_sc=False),
        out_type=jax.ShapeDtypeStruct((_N, DIM), jnp.float32),
        scratch_types=(
            [pltpu.VMEM((_PER_W,), jnp.int32)]
            + [pltpu.VMEM((_C, DIM), jnp.float32)] * _NBUF
            + [pltpu.SemaphoreType.DMA] * (2 * _NBUF)
        ),
    )
    return f(table, xflat)


def kernel(x, emb_weight):
    xflat = x.astype(jnp.int32).reshape(_N)
    out = _emb(xflat, emb_weight)
    return out.reshape(BATCH, HIST, DIM)


# speedup vs baseline: 1.6239x; 1.6239x over previous
import jax
import jax.numpy as jnp
from jax import lax
from jax.experimental import pallas as pl
from jax.experimental.pallas import tpu as pltpu
from jax.experimental.pallas import tpu_sc as plsc

NUM_TOKENS = 1000000
DIM = 32
BATCH = 16384
HIST = 50

_N = BATCH * HIST
_NW = 32
_PER_W = _N // _NW
_CB = 16
_C = _CB * HIST
_BPW = BATCH // _NW
_NSTEPS = _BPW // _CB
_NBUF = 2


def _emb_body(table, xflat, out, idx_v, *bufs):
    rows = bufs[:_NBUF]
    gs = bufs[_NBUF:2 * _NBUF]
    os_ = bufs[2 * _NBUF:3 * _NBUF]
    c = lax.axis_index("c")
    s = lax.axis_index("s")
    wid = s * 2 + c
    base = wid * _PER_W
    bbase = wid * _BPW
    pltpu.sync_copy(xflat.at[pl.ds(base, _PER_W)], idx_v)

    gh = [None] * _NBUF
    oh = [[] for _ in range(_NBUF)]

    def start(step, b):
        off = step * _C
        gh[b] = pltpu.async_copy(table.at[idx_v.at[pl.ds(off, _C)]],
                                 rows[b], gs[b])

    start(0, 0)
    for i in range(_NSTEPS):
        b = i % _NBUF
        nb = 1 - b
        if i >= 1:
            for h in oh[nb]:
                h.wait()
        if i + 1 < _NSTEPS:
            start(i + 1, nb)
        gh[b].wait()
        oh[b] = [
            pltpu.async_copy(rows[b].at[pl.ds(j * HIST, HIST)],
                             out.at[bbase + i * _CB + j], os_[b])
            for j in range(_CB)
        ]
    for h in oh[(_NSTEPS - 1) % _NBUF]:
        h.wait()


@jax.jit
def _emb(xflat, table):
    mesh = plsc.VectorSubcoreMesh(core_axis_name="c", subcore_axis_name="s")
    f = pl.kernel(
        _emb_body,
        mesh=mesh,
        compiler_params=pltpu.CompilerParams(use_tc_tiling_on_sc=False),
        out_type=jax.ShapeDtypeStruct((BATCH, HIST, DIM), jnp.float32),
        scratch_types=(
            [pltpu.VMEM((_PER_W,), jnp.int32)]
            + [pltpu.VMEM((_C, DIM), jnp.float32)] * _NBUF
            + [pltpu.SemaphoreType.DMA] * (2 * _NBUF)
        ),
    )
    return f(table, xflat)


def kernel(x, emb_weight):
    xflat = x.astype(jnp.int32).reshape(_N)
    return _emb(xflat, emb_weight)


# speedup vs baseline: 1.7443x; 1.0742x over previous
import jax
import jax.numpy as jnp
from jax import lax
from jax.experimental import pallas as pl
from jax.experimental.pallas import tpu as pltpu
from jax.experimental.pallas import tpu_sc as plsc

NUM_TOKENS = 1000000
DIM = 32
BATCH = 16384
HIST = 50

_NW = 32
_BPW = BATCH // _NW
_NBUF = 2


def _emb_body(table, xT, out, idx_v, rows0, rows1, gs0, gs1, os0, os1):
    c = lax.axis_index("c")
    s = lax.axis_index("s")
    wid = s * 2 + c
    b0 = wid * _BPW
    pltpu.sync_copy(xT.at[:, pl.ds(b0, _BPW)], idx_v)

    rows = (rows0, rows1)
    gs = (gs0, gs1)
    os_ = (os0, os1)
    gh = [None, None]
    oh = [None, None]

    def start(h, b):
        gh[b] = pltpu.async_copy(table.at[idx_v.at[h]], rows[b], gs[b])

    start(0, 0)
    for h in range(HIST):
        b = h % 2
        nb = 1 - b
        if h >= 1:
            oh[nb].wait()
        if h + 1 < HIST:
            start(h + 1, nb)
        gh[b].wait()
        oh[b] = pltpu.async_copy(rows[b], out.at[h, pl.ds(b0, _BPW)],
                                 os_[b])
    oh[(HIST - 1) % 2].wait()


@jax.jit
def _emb(xT, table):
    mesh = plsc.VectorSubcoreMesh(core_axis_name="c", subcore_axis_name="s")
    f = pl.kernel(
        _emb_body,
        mesh=mesh,
        compiler_params=pltpu.CompilerParams(use_tc_tiling_on_sc=False),
        out_type=jax.ShapeDtypeStruct((HIST, BATCH, DIM), jnp.float32),
        scratch_types=(
            [pltpu.VMEM((HIST, _BPW), jnp.int32)]
            + [pltpu.VMEM((_BPW, DIM), jnp.float32)] * _NBUF
            + [pltpu.SemaphoreType.DMA] * (2 * _NBUF)
        ),
    )
    return f(table, xT)


def kernel(x, emb_weight):
    xT = x.astype(jnp.int32).T
    out5 = _emb(xT, emb_weight)
    return jnp.transpose(out5, (1, 0, 2))
